# baseline (speedup 1.0000x reference)
import functools

import jax
import jax.numpy as jnp
from jax import lax
from jax.experimental import pallas as pl
from jax.experimental.pallas import tpu as pltpu
from jax.experimental.pallas import tpu_sc as plsc

_NUM_CORES = 2
_NUM_SUBCORES = 16
_NW = _NUM_CORES * _NUM_SUBCORES
_CHUNK_MAX = 120


def _chunk_schedule(span):
    sched = []
    off = 0
    while off < span:
        c = min(_CHUNK_MAX, span - off)
        sched.append((off, c))
        off += c
    return tuple(sched)


@functools.lru_cache(maxsize=None)
def _make_gather(b, s, v, d):
    span = (b * s) // _NW
    per_row = s // span
    sched = _chunk_schedule(span)
    n_chunks = len(sched)
    mesh = plsc.VectorSubcoreMesh(core_axis_name="c", subcore_axis_name="s")

    @functools.partial(
        pl.kernel,
        mesh=mesh,
        out_type=jax.ShapeDtypeStruct((b, s, d), jnp.float32),
        scratch_types=[
            pltpu.VMEM((span,), jnp.int32),
            pltpu.VMEM((_CHUNK_MAX, d), jnp.float32),
            pltpu.SemaphoreType.DMA,
            pltpu.SemaphoreType.DMA,
        ],
    )
    def gather_kernel(idx_hbm, table_hbm, out_hbm, idx_v, rows_v, gsem, osem):
        wid = lax.axis_index("s") * _NUM_CORES + lax.axis_index("c")
        row = wid // per_row
        col0 = (wid % per_row) * span
        pltpu.sync_copy(idx_hbm.at[row, pl.ds(col0, span)], idx_v)

        oc = None
        for j in range(n_chunks):
            off, c = sched[j]
            if oc is not None:
                oc.wait()
            pltpu.async_copy(
                table_hbm.at[idx_v.at[pl.ds(off, c)]],
                rows_v.at[pl.ds(0, c)], gsem).wait()
            oc = pltpu.async_copy(
                rows_v.at[pl.ds(0, c)],
                out_hbm.at[row, pl.ds(col0 + off, c)], osem)
        oc.wait()

    return gather_kernel


def kernel(x, table):
    b, s = x.shape
    v, d = table.shape
    return _make_gather(b, s, v, d)(x.astype(jnp.int32), table)


# speedup vs baseline: 1.0169x; 1.0169x over previous
import functools

import jax
import jax.numpy as jnp
from jax import lax
from jax.experimental import pallas as pl
from jax.experimental.pallas import tpu as pltpu
from jax.experimental.pallas import tpu_sc as plsc

_NUM_CORES = 2
_NUM_SUBCORES = 16
_NW = _NUM_CORES * _NUM_SUBCORES
_CHUNK_MAX = 56


def _chunk_schedule(span):
    sched = []
    off = 0
    while off < span:
        c = min(_CHUNK_MAX, span - off)
        sched.append((off, c))
        off += c
    return tuple(sched)


@functools.lru_cache(maxsize=None)
def _make_gather(b, s, v, d):
    span = (b * s) // _NW
    per_row = s // span
    sched = _chunk_schedule(span)
    n_chunks = len(sched)
    mesh = plsc.VectorSubcoreMesh(core_axis_name="c", subcore_axis_name="s")

    @functools.partial(
        pl.kernel,
        mesh=mesh,
        out_type=jax.ShapeDtypeStruct((b, s, d), jnp.float32),
        scratch_types=[
            pltpu.VMEM((span,), jnp.int32),
            pltpu.VMEM((_CHUNK_MAX, d), jnp.float32),
            pltpu.VMEM((_CHUNK_MAX, d), jnp.float32),
            pltpu.SemaphoreType.DMA,
            pltpu.SemaphoreType.DMA,
            pltpu.SemaphoreType.DMA,
            pltpu.SemaphoreType.DMA,
        ],
    )
    def gather_kernel(idx_hbm, table_hbm, out_hbm, idx_v, rows0, rows1,
                      g0, g1, o0, o1):
        rows = (rows0, rows1)
        gsem = (g0, g1)
        osem = (o0, o1)
        wid = lax.axis_index("s") * _NUM_CORES + lax.axis_index("c")
        row = wid // per_row
        col0 = (wid % per_row) * span
        pltpu.sync_copy(idx_hbm.at[row, pl.ds(col0, span)], idx_v)

        def gather_start(j, bnum):
            off, c = sched[j]
            return pltpu.async_copy(
                table_hbm.at[idx_v.at[pl.ds(off, c)]],
                rows[bnum].at[pl.ds(0, c)], gsem[bnum])

        def out_start(j, bnum):
            off, c = sched[j]
            return pltpu.async_copy(
                rows[bnum].at[pl.ds(0, c)],
                out_hbm.at[row, pl.ds(col0 + off, c)], osem[bnum])

        gathers = {}
        outs = {}
        for j in range(min(2, n_chunks)):
            gathers[j] = gather_start(j, j % 2)
        for j in range(n_chunks):
            bnum = j % 2
            gathers[j].wait()
            outs[j] = out_start(j, bnum)
            nj = j + 2
            if nj < n_chunks:
                outs[j].wait()
                gathers[nj] = gather_start(nj, bnum)
        for j in range(max(0, n_chunks - 2), n_chunks):
            outs[j].wait()

    return gather_kernel


def kernel(x, table):
    b, s = x.shape
    v, d = table.shape
    return _make_gather(b, s, v, d)(x.astype(jnp.int32), table)
